# baseline (speedup 1.0000x reference)
import jax
import jax.numpy as jnp
from jax import lax
from jax.experimental import pallas as pl
from jax.experimental.pallas import tpu as pltpu
from jax.experimental.pallas import tpu_sc as plsc

HIDDEN = 128
B = 1024
NEG = 200
GAMMA = 12.0
HIER_W = 0.1

NC = 2
NS = 16
L = 16
NW = NC * NS
BPW = B // NW
HC = HIDDEN // L
NG = (NEG + L - 1) // L
CHUNKS = ((0, 64, 0, 4), (64, 48, 4, 7), (112, 48, 7, 10), (160, 40, 10, 13))


def _score_kernel(pos0_hbm, pos1_hbm, neg_hbm, es_hbm, ed_hbm, rel_hbm,
                  rdir_hbm, hier_hbm, out_hbm,
                  pos0_v, pos1_v, neg_v, hs_v, hd_v, rl_v, rd_v, hl_v,
                  ts_v, td_v, tl_v, out_v, sem0, sem1, sem2, sem3):
    _LANE = lax.iota(jnp.int32, L)
    _ZERO = jnp.zeros((L,), jnp.int32)
    sems = (sem0, sem1, sem2, sem3)
    wid = lax.axis_index("s") * NC + lax.axis_index("c")
    base = wid * BPW

    pltpu.sync_copy(neg_hbm.at[pl.ds(base, BPW)], neg_v)
    pltpu.sync_copy(pos0_hbm.at[pl.ds(base, BPW)], pos0_v)
    pltpu.sync_copy(pos1_hbm.at[pl.ds(base, BPW)], pos1_v)

    head_cps = [
        pltpu.async_copy(es_hbm.at[pos0_v], hs_v, sem0),
        pltpu.async_copy(ed_hbm.at[pos0_v], hd_v, sem0),
        pltpu.async_copy(rel_hbm.at[pos1_v], rl_v, sem0),
        pltpu.async_copy(rdir_hbm.at[pos1_v], rd_v, sem0),
        pltpu.async_copy(hier_hbm.at[pos0_v], hl_v, sem0),
    ]
    for cp in head_cps:
        cp.wait()

    def b_body(b, carry):
        cps = []
        for ci, (off, sz, _, _) in enumerate(CHUNKS):
            idx = neg_v.at[b, pl.ds(off, sz)]
            cps.append((
                pltpu.async_copy(es_hbm.at[idx], ts_v.at[pl.ds(off, sz)], sems[ci]),
                pltpu.async_copy(ed_hbm.at[idx], td_v.at[pl.ds(off, sz)], sems[ci]),
                pltpu.async_copy(hier_hbm.at[idx], tl_v.at[pl.ds(off, sz)], sems[ci]),
            ))

        bsplat = jnp.full((L,), b, jnp.int32)
        qs = [hs_v[b, pl.ds(c * L, L)] + rl_v[b, pl.ds(c * L, L)] for c in range(HC)]
        qd = [hd_v[b, pl.ds(c * L, L)] for c in range(HC)]
        c1v = HIER_W * plsc.load_gather(rd_v, [bsplat, _ZERO])
        c0v = GAMMA - c1v * plsc.load_gather(hl_v, [bsplat, _ZERO])

        def g_body(g, gcarry):
            n0 = g * L
            nidx = jnp.minimum(_LANE + n0, NEG - 1)
            tl_g = plsc.load_gather(tl_v, [nidx, _ZERO])
            svec = jnp.zeros((L,), jnp.float32)
            for j in range(L):
                n = jnp.minimum(n0 + j, NEG - 1)
                acc = jnp.abs(qs[0] - ts_v[n, pl.ds(0, L)])
                acc = acc + jnp.abs(qd[0] - td_v[n, pl.ds(0, L)])
                for c in range(1, HC):
                    acc = acc + jnp.abs(qs[c] - ts_v[n, pl.ds(c * L, L)])
                    acc = acc + jnp.abs(qd[c] - td_v[n, pl.ds(c * L, L)])
                s = jnp.sum(acc)
                svec = jnp.where(_LANE == j, s, svec)
            score = c0v - svec + c1v * tl_g
            plsc.store_scatter(out_v, [bsplat, _LANE + n0], score,
                               mask=(_LANE + n0) < NEG)
            return gcarry

        for ci, (_, _, g0, g1) in enumerate(CHUNKS):
            for cp in cps[ci]:
                cp.wait()
            lax.fori_loop(g0, g1, g_body, 0)
        return carry

    lax.fori_loop(0, BPW, b_body, 0)
    pltpu.sync_copy(out_v, out_hbm.at[pl.ds(base, BPW)])


@jax.jit
def kernel(positive_sample, negative_sample, entity_static, entity_dynamic,
           relation_emb, r_direction, entity_hierarchy):
    pos0 = positive_sample[:, 0]
    pos1 = positive_sample[:, 1]
    rdir_p = jnp.pad(r_direction, ((0, 0), (0, L - 1)))
    hier_p = jnp.pad(entity_hierarchy, ((0, 0), (0, L - 1)))
    mesh = plsc.VectorSubcoreMesh(core_axis_name="c", subcore_axis_name="s")
    f = pl.kernel(
        _score_kernel,
        mesh=mesh,
        compiler_params=pltpu.CompilerParams(use_tc_tiling_on_sc=False,
                                             needs_layout_passes=False),
        out_type=jax.ShapeDtypeStruct((B, NEG), jnp.float32),
        scratch_types=[
            pltpu.VMEM((BPW,), jnp.int32),
            pltpu.VMEM((BPW,), jnp.int32),
            pltpu.VMEM((BPW, NEG), jnp.int32),
            pltpu.VMEM((BPW, HIDDEN), jnp.float32),
            pltpu.VMEM((BPW, HIDDEN), jnp.float32),
            pltpu.VMEM((BPW, HIDDEN), jnp.float32),
            pltpu.VMEM((BPW, L), jnp.float32),
            pltpu.VMEM((BPW, L), jnp.float32),
            pltpu.VMEM((NEG, HIDDEN), jnp.float32),
            pltpu.VMEM((NEG, HIDDEN), jnp.float32),
            pltpu.VMEM((NEG, L), jnp.float32),
            pltpu.VMEM((BPW, NEG), jnp.float32),
            pltpu.SemaphoreType.DMA,
            pltpu.SemaphoreType.DMA,
            pltpu.SemaphoreType.DMA,
            pltpu.SemaphoreType.DMA,
        ],
    )
    return f(pos0, pos1, negative_sample, entity_static, entity_dynamic,
             relation_emb, rdir_p, hier_p)


# speedup vs baseline: 1.0218x; 1.0218x over previous
import jax
import jax.numpy as jnp
from jax import lax
from jax.experimental import pallas as pl
from jax.experimental.pallas import tpu as pltpu
from jax.experimental.pallas import tpu_sc as plsc

HIDDEN = 128
B = 1024
NEG = 200
GAMMA = 12.0
HIER_W = 0.1

NC = 2
NS = 16
L = 16
NW = NC * NS
BPW = B // NW
CHUNKS = ((0, 104), (104, 96))
HC = HIDDEN // L
NG = (NEG + L - 1) // L

def _score_kernel(pos0_hbm, pos1_hbm, neg_hbm, es_hbm, ed_hbm, rel_hbm,
                  rdir_hbm, hier_hbm, out_hbm,
                  pos0_v, pos1_v, neg_v, hs_v, hd_v, rl_v, rd_v, hl_v,
                  ts_v, td_v, tl_v, out_v, sem):
    _LANE = lax.iota(jnp.int32, L)
    _ZERO = jnp.zeros((L,), jnp.int32)
    wid = lax.axis_index("s") * NC + lax.axis_index("c")
    base = wid * BPW

    pltpu.sync_copy(neg_hbm.at[pl.ds(base, BPW)], neg_v)
    pltpu.sync_copy(pos0_hbm.at[pl.ds(base, BPW)], pos0_v)
    pltpu.sync_copy(pos1_hbm.at[pl.ds(base, BPW)], pos1_v)

    head_cps = [
        pltpu.async_copy(es_hbm.at[pos0_v], hs_v, sem),
        pltpu.async_copy(ed_hbm.at[pos0_v], hd_v, sem),
        pltpu.async_copy(rel_hbm.at[pos1_v], rl_v, sem),
        pltpu.async_copy(rdir_hbm.at[pos1_v], rd_v, sem),
        pltpu.async_copy(hier_hbm.at[pos0_v], hl_v, sem),
    ]
    for cp in head_cps:
        cp.wait()

    def b_body(b, carry):
        cps = []
        for off, sz in CHUNKS:
            idx = neg_v.at[b, pl.ds(off, sz)]
            cps.append(pltpu.async_copy(es_hbm.at[idx], ts_v.at[pl.ds(off, sz)], sem))
            cps.append(pltpu.async_copy(ed_hbm.at[idx], td_v.at[pl.ds(off, sz)], sem))
            cps.append(pltpu.async_copy(hier_hbm.at[idx], tl_v.at[pl.ds(off, sz)], sem))
        for cp in cps:
            cp.wait()

        bsplat = jnp.full((L,), b, jnp.int32)
        qs = [hs_v[b, pl.ds(c * L, L)] + rl_v[b, pl.ds(c * L, L)] for c in range(HC)]
        qd = [hd_v[b, pl.ds(c * L, L)] for c in range(HC)]
        c1v = HIER_W * plsc.load_gather(rd_v, [bsplat, _ZERO])
        c0v = GAMMA - c1v * plsc.load_gather(hl_v, [bsplat, _ZERO])

        def g_body(g, gcarry):
            n0 = g * L
            nidx = jnp.minimum(_LANE + n0, NEG - 1)
            tl_g = plsc.load_gather(tl_v, [nidx, _ZERO])
            svec = jnp.zeros((L,), jnp.float32)
            for j in range(L):
                n = jnp.minimum(n0 + j, NEG - 1)
                acc = jnp.abs(qs[0] - ts_v[n, pl.ds(0, L)])
                acc = acc + jnp.abs(qd[0] - td_v[n, pl.ds(0, L)])
                for c in range(1, HC):
                    acc = acc + jnp.abs(qs[c] - ts_v[n, pl.ds(c * L, L)])
                    acc = acc + jnp.abs(qd[c] - td_v[n, pl.ds(c * L, L)])
                s = jnp.sum(acc)
                svec = jnp.where(_LANE == j, s, svec)
            score = c0v - svec + c1v * tl_g
            plsc.store_scatter(out_v, [bsplat, _LANE + n0], score,
                               mask=(_LANE + n0) < NEG)
            return gcarry

        lax.fori_loop(0, NG, g_body, 0)
        return carry

    lax.fori_loop(0, BPW, b_body, 0)
    pltpu.sync_copy(out_v, out_hbm.at[pl.ds(base, BPW)])


@jax.jit
def kernel(positive_sample, negative_sample, entity_static, entity_dynamic,
           relation_emb, r_direction, entity_hierarchy):
    pos0 = positive_sample[:, 0]
    pos1 = positive_sample[:, 1]
    rdir_p = jnp.pad(r_direction, ((0, 0), (0, L - 1)))
    hier_p = jnp.pad(entity_hierarchy, ((0, 0), (0, L - 1)))
    mesh = plsc.VectorSubcoreMesh(core_axis_name="c", subcore_axis_name="s")
    f = pl.kernel(
        _score_kernel,
        mesh=mesh,
        compiler_params=pltpu.CompilerParams(use_tc_tiling_on_sc=False,
                                              needs_layout_passes=False),
        out_type=jax.ShapeDtypeStruct((B, NEG), jnp.float32),
        scratch_types=[
            pltpu.VMEM((BPW,), jnp.int32),
            pltpu.VMEM((BPW,), jnp.int32),
            pltpu.VMEM((BPW, NEG), jnp.int32),
            pltpu.VMEM((BPW, HIDDEN), jnp.float32),
            pltpu.VMEM((BPW, HIDDEN), jnp.float32),
            pltpu.VMEM((BPW, HIDDEN), jnp.float32),
            pltpu.VMEM((BPW, L), jnp.float32),
            pltpu.VMEM((BPW, L), jnp.float32),
            pltpu.VMEM((NEG, HIDDEN), jnp.float32),
            pltpu.VMEM((NEG, HIDDEN), jnp.float32),
            pltpu.VMEM((NEG, L), jnp.float32),
            pltpu.VMEM((BPW, NEG), jnp.float32),
            pltpu.SemaphoreType.DMA,
        ],
    )
    return f(pos0, pos1, negative_sample, entity_static, entity_dynamic,
             relation_emb, rdir_p, hier_p)


# speedup vs baseline: 1.4491x; 1.4181x over previous
import jax
import jax.numpy as jnp
from jax import lax
from jax.experimental import pallas as pl
from jax.experimental.pallas import tpu as pltpu
from jax.experimental.pallas import tpu_sc as plsc

HIDDEN = 128
B = 1024
NEG = 200
GAMMA = 12.0
HIER_W = 0.1

NC = 2
NS = 16
L = 16
NW = NC * NS
BPW = B // NW
CHUNKS = ((0, 104), (104, 96))
HC = HIDDEN // L
NG = (NEG + L - 1) // L
SPANS = tuple(range(0, NEG - L, L)) + (NEG - L,)


def _score_kernel(pos0_hbm, pos1_hbm, neg_hbm, es_hbm, ed_hbm, rel_hbm,
                  rdir_hbm, hier_hbm, out_hbm,
                  pos0_v, pos1_v, pl0_v, pl1_v, nl_v, neg_v, hs_v, hd_v,
                  rl_v, rd_v, hl_v, ts_v, td_v, tl_v, out_v, sem):
    _LANE = lax.iota(jnp.int32, L)
    wid = lax.axis_index("s") * NC + lax.axis_index("c")
    base = wid * BPW

    pltpu.sync_copy(neg_hbm.at[pl.ds(base, BPW)], neg_v)
    pltpu.sync_copy(pos0_hbm.at[pl.ds(base, BPW)], pos0_v)
    pltpu.sync_copy(pos1_hbm.at[pl.ds(base, BPW)], pos1_v)

    for k in range(BPW // L):
        pl0_v[pl.ds(k * L, L)] = lax.shift_right_logical(pos0_v[pl.ds(k * L, L)], 4)
        pl1_v[pl.ds(k * L, L)] = lax.shift_right_logical(pos1_v[pl.ds(k * L, L)], 4)

    head_cps = [
        pltpu.async_copy(es_hbm.at[pos0_v], hs_v, sem),
        pltpu.async_copy(ed_hbm.at[pos0_v], hd_v, sem),
        pltpu.async_copy(rel_hbm.at[pos1_v], rl_v, sem),
        pltpu.async_copy(rdir_hbm.at[pl1_v], rd_v, sem),
        pltpu.async_copy(hier_hbm.at[pl0_v], hl_v, sem),
    ]
    for cp in head_cps:
        cp.wait()

    def b_body(b, carry):
        for off in SPANS:
            nl_v[pl.ds(off, L)] = lax.shift_right_logical(
                neg_v[b, pl.ds(off, L)], 4)
        cps = []
        for off, sz in CHUNKS:
            idx = neg_v.at[b, pl.ds(off, sz)]
            cps.append(pltpu.async_copy(es_hbm.at[idx], ts_v.at[pl.ds(off, sz)], sem))
            cps.append(pltpu.async_copy(ed_hbm.at[idx], td_v.at[pl.ds(off, sz)], sem))
            cps.append(pltpu.async_copy(hier_hbm.at[nl_v.at[pl.ds(off, sz)]],
                                        tl_v.at[pl.ds(off, sz)], sem))
        for cp in cps:
            cp.wait()

        bsplat = jnp.full((L,), b, jnp.int32)
        qs = [hs_v[b, pl.ds(c * L, L)] + rl_v[b, pl.ds(c * L, L)] for c in range(HC)]
        qd = [hd_v[b, pl.ds(c * L, L)] for c in range(HC)]
        p0b = plsc.load_gather(pos0_v, [bsplat])
        p1b = plsc.load_gather(pos1_v, [bsplat])
        c1v = HIER_W * plsc.load_gather(rd_v, [bsplat, p1b & 15])
        c0v = GAMMA - c1v * plsc.load_gather(hl_v, [bsplat, p0b & 15])

        def g_body(g, gcarry):
            n0 = g * L
            nidx = jnp.minimum(_LANE + n0, NEG - 1)
            negg = plsc.load_gather(neg_v, [bsplat, nidx])
            tl_g = plsc.load_gather(tl_v, [nidx, negg & 15])
            svec = jnp.zeros((L,), jnp.float32)
            for j in range(L):
                n = jnp.minimum(n0 + j, NEG - 1)
                acc = jnp.abs(qs[0] - ts_v[n, pl.ds(0, L)])
                acc = acc + jnp.abs(qd[0] - td_v[n, pl.ds(0, L)])
                for c in range(1, HC):
                    acc = acc + jnp.abs(qs[c] - ts_v[n, pl.ds(c * L, L)])
                    acc = acc + jnp.abs(qd[c] - td_v[n, pl.ds(c * L, L)])
                s = jnp.sum(acc)
                svec = jnp.where(_LANE == j, s, svec)
            score = c0v - svec + c1v * tl_g
            plsc.store_scatter(out_v, [bsplat, _LANE + n0], score,
                               mask=(_LANE + n0) < NEG)
            return gcarry

        lax.fori_loop(0, NG, g_body, 0)
        return carry

    lax.fori_loop(0, BPW, b_body, 0)
    pltpu.sync_copy(out_v, out_hbm.at[pl.ds(base, BPW)])


@jax.jit
def kernel(positive_sample, negative_sample, entity_static, entity_dynamic,
           relation_emb, r_direction, entity_hierarchy):
    pos0 = positive_sample[:, 0]
    pos1 = positive_sample[:, 1]
    hier_r = entity_hierarchy.reshape(-1, L)
    rdir_r = jnp.pad(r_direction[:, 0], (0, 512 - 500)).reshape(-1, L)
    mesh = plsc.VectorSubcoreMesh(core_axis_name="c", subcore_axis_name="s")
    f = pl.kernel(
        _score_kernel,
        mesh=mesh,
        compiler_params=pltpu.CompilerParams(use_tc_tiling_on_sc=False,
                                             needs_layout_passes=False),
        out_type=jax.ShapeDtypeStruct((B, NEG), jnp.float32),
        scratch_types=[
            pltpu.VMEM((BPW,), jnp.int32),
            pltpu.VMEM((BPW,), jnp.int32),
            pltpu.VMEM((BPW,), jnp.int32),
            pltpu.VMEM((BPW,), jnp.int32),
            pltpu.VMEM((NEG,), jnp.int32),
            pltpu.VMEM((BPW, NEG), jnp.int32),
            pltpu.VMEM((BPW, HIDDEN), jnp.float32),
            pltpu.VMEM((BPW, HIDDEN), jnp.float32),
            pltpu.VMEM((BPW, HIDDEN), jnp.float32),
            pltpu.VMEM((BPW, L), jnp.float32),
            pltpu.VMEM((BPW, L), jnp.float32),
            pltpu.VMEM((NEG, HIDDEN), jnp.float32),
            pltpu.VMEM((NEG, HIDDEN), jnp.float32),
            pltpu.VMEM((NEG, L), jnp.float32),
            pltpu.VMEM((BPW, NEG), jnp.float32),
            pltpu.SemaphoreType.DMA,
        ],
    )
    return f(pos0, pos1, negative_sample, entity_static, entity_dynamic,
             relation_emb, rdir_r, hier_r)


# speedup vs baseline: 1.4737x; 1.0170x over previous
import jax
import jax.numpy as jnp
from jax import lax
from jax.experimental import pallas as pl
from jax.experimental.pallas import tpu as pltpu
from jax.experimental.pallas import tpu_sc as plsc

HIDDEN = 128
B = 1024
NEG = 200
GAMMA = 12.0
HIER_W = 0.1

NC = 2
NS = 16
L = 16
NW = NC * NS
BPW = B // NW
CHUNKS = ((0, 104), (104, 96))
HC = HIDDEN // L
NG = (NEG + L - 1) // L
SPANS = tuple(range(0, NEG - L, L)) + (NEG - L,)


def _score_kernel(pos0_hbm, pos1_hbm, neg_hbm, es_hbm, ed_hbm, rel_hbm,
                  rdir_hbm, hier_hbm, out_hbm,
                  pos0_v, pos1_v, pl0_v, pl1_v, nl_v, neg_v, hs_v, hd_v,
                  rl_v, rd_v, hl_v, ts_v, td_v, tl_v, out_v, sem):
    _LANE = lax.iota(jnp.int32, L)
    wid = lax.axis_index("s") * NC + lax.axis_index("c")
    base = wid * BPW

    pltpu.sync_copy(neg_hbm.at[pl.ds(base, BPW)], neg_v)
    pltpu.sync_copy(pos0_hbm.at[pl.ds(base, BPW)], pos0_v)
    pltpu.sync_copy(pos1_hbm.at[pl.ds(base, BPW)], pos1_v)

    for k in range(BPW // L):
        pl0_v[pl.ds(k * L, L)] = lax.shift_right_logical(pos0_v[pl.ds(k * L, L)], 4)
        pl1_v[pl.ds(k * L, L)] = lax.shift_right_logical(pos1_v[pl.ds(k * L, L)], 4)

    head_cps = [
        pltpu.async_copy(es_hbm.at[pos0_v], hs_v, sem),
        pltpu.async_copy(ed_hbm.at[pos0_v], hd_v, sem),
        pltpu.async_copy(rel_hbm.at[pos1_v], rl_v, sem),
        pltpu.async_copy(rdir_hbm.at[pl1_v], rd_v, sem),
        pltpu.async_copy(hier_hbm.at[pl0_v], hl_v, sem),
    ]
    for cp in head_cps:
        cp.wait()

    def b_body(b, carry):
        for off in SPANS:
            nl_v[pl.ds(off, L)] = lax.shift_right_logical(
                neg_v[b, pl.ds(off, L)], 4)
        cps = []
        for off, sz in CHUNKS:
            idx = neg_v.at[b, pl.ds(off, sz)]
            cps.append(pltpu.async_copy(es_hbm.at[idx], ts_v.at[pl.ds(off, sz)], sem))
            cps.append(pltpu.async_copy(ed_hbm.at[idx], td_v.at[pl.ds(off, sz)], sem))
            cps.append(pltpu.async_copy(hier_hbm.at[nl_v.at[pl.ds(off, sz)]],
                                        tl_v.at[pl.ds(off, sz)], sem))
        for cp in cps:
            cp.wait()

        bsplat = jnp.full((L,), b, jnp.int32)
        qs = [hs_v[b, pl.ds(c * L, L)] + rl_v[b, pl.ds(c * L, L)] for c in range(HC)]
        qd = [hd_v[b, pl.ds(c * L, L)] for c in range(HC)]
        p0b = plsc.load_gather(pos0_v, [bsplat])
        p1b = plsc.load_gather(pos1_v, [bsplat])
        c1v = HIER_W * plsc.load_gather(rd_v, [bsplat, p1b & 15])
        c0v = GAMMA - c1v * plsc.load_gather(hl_v, [bsplat, p0b & 15])

        def group_score(ns, nidx):
            negg = plsc.load_gather(neg_v, [bsplat, nidx])
            tl_g = plsc.load_gather(tl_v, [nidx, negg & 15])
            svec = jnp.zeros((L,), jnp.float32)
            for j in range(L):
                n = ns[j]
                acc = jnp.abs(qs[0] - ts_v[n, pl.ds(0, L)])
                acc = acc + jnp.abs(qd[0] - td_v[n, pl.ds(0, L)])
                for c in range(1, HC):
                    acc = acc + jnp.abs(qs[c] - ts_v[n, pl.ds(c * L, L)])
                    acc = acc + jnp.abs(qd[c] - td_v[n, pl.ds(c * L, L)])
                s = jnp.sum(acc)
                svec = jnp.where(_LANE == j, s, svec)
            return c0v - svec + c1v * tl_g

        def g_full(g, gcarry):
            n0 = g * L
            score = group_score([n0 + j for j in range(L)], _LANE + n0)
            plsc.store_scatter(out_v, [bsplat, _LANE + n0], score)
            return gcarry

        lax.fori_loop(0, NG - 1, g_full, 0)
        n0p = (NG - 1) * L
        scorep = group_score([min(n0p + j, NEG - 1) for j in range(L)],
                             jnp.minimum(_LANE + n0p, NEG - 1))
        plsc.store_scatter(out_v, [bsplat, _LANE + n0p], scorep,
                           mask=(_LANE + n0p) < NEG)
        return carry

    lax.fori_loop(0, BPW, b_body, 0)
    pltpu.sync_copy(out_v, out_hbm.at[pl.ds(base, BPW)])


@jax.jit
def kernel(positive_sample, negative_sample, entity_static, entity_dynamic,
           relation_emb, r_direction, entity_hierarchy):
    pos0 = positive_sample[:, 0]
    pos1 = positive_sample[:, 1]
    hier_r = entity_hierarchy.reshape(-1, L)
    rdir_r = jnp.pad(r_direction[:, 0], (0, 512 - 500)).reshape(-1, L)
    mesh = plsc.VectorSubcoreMesh(core_axis_name="c", subcore_axis_name="s")
    f = pl.kernel(
        _score_kernel,
        mesh=mesh,
        compiler_params=pltpu.CompilerParams(use_tc_tiling_on_sc=False,
                                             needs_layout_passes=False),
        out_type=jax.ShapeDtypeStruct((B, NEG), jnp.float32),
        scratch_types=[
            pltpu.VMEM((BPW,), jnp.int32),
            pltpu.VMEM((BPW,), jnp.int32),
            pltpu.VMEM((BPW,), jnp.int32),
            pltpu.VMEM((BPW,), jnp.int32),
            pltpu.VMEM((NEG,), jnp.int32),
            pltpu.VMEM((BPW, NEG), jnp.int32),
            pltpu.VMEM((BPW, HIDDEN), jnp.float32),
            pltpu.VMEM((BPW, HIDDEN), jnp.float32),
            pltpu.VMEM((BPW, HIDDEN), jnp.float32),
            pltpu.VMEM((BPW, L), jnp.float32),
            pltpu.VMEM((BPW, L), jnp.float32),
            pltpu.VMEM((NEG, HIDDEN), jnp.float32),
            pltpu.VMEM((NEG, HIDDEN), jnp.float32),
            pltpu.VMEM((NEG, L), jnp.float32),
            pltpu.VMEM((BPW, NEG), jnp.float32),
            pltpu.SemaphoreType.DMA,
        ],
    )
    return f(pos0, pos1, negative_sample, entity_static, entity_dynamic,
             relation_emb, rdir_r, hier_r)
